# baseline (device time: 15262 ns/iter reference)
import jax
import jax.numpy as jnp
from jax import lax
from jax.experimental import pallas as pl
from jax.experimental.pallas import tpu as pltpu

N_DEV = 32
K = 8


def _topk_rows(c, k):
    ms = []
    for _ in range(k):
        m = jnp.max(c, axis=0, keepdims=True)
        ms.append(m)
        c = jnp.where(c == m, -jnp.inf, c)
    return jnp.concatenate(ms, axis=0)


def kernel(x):
    m_rows, n_local = x.shape

    def body(x_ref, out_ref, comm_ref, send_sems, recv_sems):
        me = lax.axis_index("i")

        barrier_sem = pltpu.get_barrier_semaphore()
        for d in range(1, N_DEV):
            t = lax.rem(me + d, N_DEV)
            pl.semaphore_signal(
                barrier_sem, inc=1,
                device_id=(t,), device_id_type=pl.DeviceIdType.MESH,
            )

        c = x_ref[...]
        ms = []
        for _ in range(K):
            m = jnp.max(c, axis=1, keepdims=True)
            ms.append(m)
            c = jnp.where(c == m, -jnp.inf, c)
        local_top = jnp.concatenate(ms, axis=1)

        comm_ref[0, :, :] = jnp.transpose(local_top)

        pl.semaphore_wait(barrier_sem, N_DEV - 1)

        rdmas = []
        for d in range(1, N_DEV):
            t = lax.rem(me + d, N_DEV)
            rdma = pltpu.make_async_remote_copy(
                src_ref=comm_ref.at[0],
                dst_ref=comm_ref.at[d],
                send_sem=send_sems.at[d],
                recv_sem=recv_sems.at[d],
                device_id=(t,),
                device_id_type=pl.DeviceIdType.MESH,
            )
            rdma.start()
            rdmas.append(rdma)
        for rdma in rdmas:
            rdma.wait_recv()
        for rdma in rdmas:
            rdma.wait_send()

        g = comm_ref[...].reshape(N_DEV * K, m_rows)
        merged_t = _topk_rows(g, K)

        out_ref[...] = jnp.transpose(merged_t)

    return pl.pallas_call(
        body,
        out_shape=jax.ShapeDtypeStruct((m_rows, K), jnp.float32),
        in_specs=[pl.BlockSpec(memory_space=pltpu.VMEM)],
        out_specs=pl.BlockSpec(memory_space=pltpu.VMEM),
        scratch_shapes=[
            pltpu.VMEM((N_DEV, K, m_rows), jnp.float32),
            pltpu.SemaphoreType.DMA((N_DEV,)),
            pltpu.SemaphoreType.DMA((N_DEV,)),
        ],
        compiler_params=pltpu.CompilerParams(collective_id=0),
    )(x)
